# baseline (speedup 1.0000x reference)
import functools

import jax
import jax.numpy as jnp
import numpy as np
from jax import lax
from jax.experimental import pallas as pl
from jax.experimental.pallas import tpu as pltpu
from jax.experimental.pallas import tpu_sc as plsc

ZERO = 1e-06
ETA_VAR = 1.0
SGLD_A, SGLD_B, SGLD_C = 0.01, 1.0, 0.55

_K2_HI, _K2_LO = 64467757, 2916123636


def _u32(v):
    return jnp.uint32(np.uint32(v))


def _threefry_bits(cnt):
    ks0 = _K2_HI
    ks1 = _K2_LO
    ks2 = ks0 ^ ks1 ^ 0x1BD11BDA
    x0 = jnp.full_like(cnt, _u32(ks0))
    x1 = cnt + _u32(ks1)
    rots = ((13, 15, 26, 6), (17, 29, 16, 24))
    keysched = ((ks1, ks2, 1), (ks2, ks0, 2), (ks0, ks1, 3),
                (ks1, ks2, 4), (ks2, ks0, 5))
    for i, (ka, kb, c) in enumerate(keysched):
        for r in rots[i % 2]:
            x0 = x0 + x1
            x1 = (x1 << _u32(r)) | (x1 >> _u32(32 - r))
            x1 = x1 ^ x0
        x0 = x0 + _u32(ka)
        x1 = x1 + _u32((kb + c) & 0xFFFFFFFF)
    return x0 ^ x1


def _sc_gather(phi, words2d):
    R, CH = words2d.shape
    Dm = phi.shape[1]
    B = R * CH
    info = plsc.get_sparse_core_info()
    NW = info.num_cores * info.num_subcores
    n_ch = R // NW
    mesh = plsc.VectorSubcoreMesh(core_axis_name="c", subcore_axis_name="s")

    @functools.partial(
        pl.kernel,
        mesh=mesh,
        compiler_params=pltpu.CompilerParams(use_tc_tiling_on_sc=False),
        out_type=jax.ShapeDtypeStruct((B, Dm), jnp.float32),
        scratch_types=[
            pltpu.VMEM((n_ch, CH), jnp.int32),
            pltpu.VMEM((CH, Dm), jnp.float32),
            pltpu.VMEM((CH, Dm), jnp.float32),
            pltpu.SemaphoreType.DMA,
            pltpu.SemaphoreType.DMA,
        ],
    )
    def k(idx_hbm, table_hbm, out_hbm, idx_v, rows_a, rows_b, sem_a, sem_b):
        wid = lax.axis_index("s") * info.num_cores + lax.axis_index("c")
        pltpu.sync_copy(idx_hbm.at[pl.ds(wid * n_ch, n_ch)], idx_v)
        obase = wid * n_ch * CH

        def gather(j, buf, sem):
            return pltpu.make_async_copy(table_hbm.at[idx_v.at[j]], buf, sem)

        gather(0, rows_a, sem_a).start()

        def body(t, _):
            j0 = 2 * t
            j1 = j0 + 1
            out2 = out_hbm
            gather(j1, rows_b, sem_b).start()
            gather(j0, rows_a, sem_a).wait()
            pltpu.sync_copy(rows_a, out2.at[pl.ds(obase + j0 * CH, CH)])

            @pl.when(j0 + 2 < n_ch)
            def _():
                gather(j0 + 2, rows_a, sem_a).start()

            gather(j1, rows_b, sem_b).wait()
            pltpu.sync_copy(rows_b, out2.at[pl.ds(obase + j1 * CH, CH)])
            return 0

        lax.fori_loop(0, n_ch // 2, body, 0)
        if n_ch % 2:
            j = n_ch - 1
            gather(j, rows_a, sem_a).wait()
            pltpu.sync_copy(rows_a, out_hbm.at[pl.ds(obase + j * CH, CH)])

    return k(words2d, phi)


def _tc_main(eta, alpha2d, CDK, xi, phi_w2, doc0=0):
    D, K = eta.shape
    NH = 100
    N = 2 * NH
    BD = 16
    eps = SGLD_A * (SGLD_B + 0.0) ** (-SGLD_C)
    eps_half = np.float32(eps / 2.0)

    def body(eta_ref, alpha_ref, cdk_ref, xi_ref, phiw_ref,
             za_ref, zb_ref, probs_ref, etan_ref, cdkn_ref):
        i = pl.program_id(0)
        eta_b = eta_ref[...]
        m = jnp.max(eta_b, axis=-1, keepdims=True)
        ex = jnp.exp(eta_b - m)
        sm = ex / jnp.sum(ex, axis=-1, keepdims=True)
        prior = (alpha_ref[...] - eta_b) / np.float32(ETA_VAR)
        grad = cdk_ref[...] - np.float32(N) * sm
        eta_new = eta_b + eps_half * (prior + grad) + xi_ref[...]
        etan_ref[...] = eta_new

        eta2 = jnp.concatenate([eta_new, eta_new], axis=-1)
        pw = phiw_ref[...].reshape(BD, NH, 128)
        logits = eta2[:, None, :] + pw

        p = jnp.maximum(jnp.exp(jnp.clip(logits, -700.0, 700.0)),
                        np.float32(ZERO))
        sa = jnp.sum(p[:, :, :K], axis=-1, keepdims=True)
        sb = jnp.sum(p[:, :, K:], axis=-1, keepdims=True)
        denom = jnp.concatenate(
            [jnp.broadcast_to(sa, (BD, NH, K)),
             jnp.broadcast_to(sb, (BD, NH, K))], axis=-1)
        probs_ref[...] = (p / denom).reshape(BD * NH, 128)

        jrow = (lax.broadcasted_iota(jnp.int32, (BD, NH, 128), 0) * NH
                + lax.broadcasted_iota(jnp.int32, (BD, NH, 128), 1))
        jl = lax.broadcasted_iota(jnp.int32, (BD, NH, 128), 2)
        cnt = ((doc0 * (NH * 128)) + (i * (BD * NH * 128))
               + jrow * 128 + jl).astype(jnp.uint32)
        bits = _threefry_bits(cnt)
        fb = lax.bitcast_convert_type(
            (bits >> _u32(9)) | _u32(0x3F800000), jnp.float32
        ) - np.float32(1.0)
        u = jnp.maximum(
            np.float32(1e-9),
            fb * (np.float32(1.0) - np.float32(1e-9)) + np.float32(1e-9),
        )
        gm = logits - jnp.log(-jnp.log(u))
        ma = jnp.max(gm[:, :, :K], axis=-1, keepdims=True)
        mb = jnp.max(gm[:, :, K:], axis=-1, keepdims=True)
        mc = jnp.concatenate(
            [jnp.broadcast_to(ma, (BD, NH, K)),
             jnp.broadcast_to(mb, (BD, NH, K))], axis=-1)
        kio = jnp.bitwise_and(jl, 63)
        cand = jnp.where(gm == mc, kio, K)
        za = jnp.min(cand[:, :, :K], axis=-1)
        zb = jnp.min(cand[:, :, K:], axis=-1)
        za_ref[...] = za
        zb_ref[...] = zb
        kio64 = lax.broadcasted_iota(jnp.int32, (BD, NH, K), 2)
        oh = ((za[:, :, None] == kio64).astype(jnp.float32)
              + (zb[:, :, None] == kio64).astype(jnp.float32))
        cdkn_ref[...] = jnp.sum(oh, axis=1)

    grid = D // BD
    return pl.pallas_call(
        body,
        grid=(grid,),
        in_specs=[
            pl.BlockSpec((BD, K), lambda i: (i, 0)),
            pl.BlockSpec((1, K), lambda i: (0, 0)),
            pl.BlockSpec((BD, K), lambda i: (i, 0)),
            pl.BlockSpec((BD, 1), lambda i: (i, 0)),
            pl.BlockSpec((BD * NH, 128), lambda i: (i, 0)),
        ],
        out_specs=[
            pl.BlockSpec((BD, NH), lambda i: (i, 0)),
            pl.BlockSpec((BD, NH), lambda i: (i, 0)),
            pl.BlockSpec((BD * NH, 128), lambda i: (i, 0)),
            pl.BlockSpec((BD, K), lambda i: (i, 0)),
            pl.BlockSpec((BD, K), lambda i: (i, 0)),
        ],
        out_shape=[
            jax.ShapeDtypeStruct((D, NH), jnp.int32),
            jax.ShapeDtypeStruct((D, NH), jnp.int32),
            jax.ShapeDtypeStruct((D * NH, 128), jnp.float32),
            jax.ShapeDtypeStruct((D, K), jnp.float32),
            jax.ShapeDtypeStruct((D, K), jnp.float32),
        ],
    )(eta, alpha2d, CDK, xi, phi_w2)


def kernel(eta, alpha, CDK, phi, words):
    D, K = eta.shape
    N = words.shape[1]
    eps = SGLD_A * (SGLD_B + 0.0) ** (-SGLD_C)
    k1 = jax.random.split(jax.random.key(42))[0]
    xi = jax.random.normal(k1, (D, 1), dtype=eta.dtype) * eps

    C = 2
    Dc = D // C
    words2d = words.reshape((D * N) // 128, 128)
    rc = words2d.shape[0] // C
    alpha2d = alpha.reshape(1, K)
    parts = []
    for c in range(C):
        w_c = lax.slice_in_dim(words2d, c * rc, (c + 1) * rc, axis=0)
        pw_c = _sc_gather(phi, w_c).reshape((Dc * N * K) // 128, 128)
        sl = lambda x: lax.slice_in_dim(x, c * Dc, (c + 1) * Dc, axis=0)
        za, zb, p2, en, ck = _tc_main(
            sl(eta), alpha2d, sl(CDK), sl(xi), pw_c, doc0=c * Dc)
        parts.append((za, zb, p2.reshape(Dc, N, K), en, ck))
    za, zb, probs, eta_new, CDK_new = (
        jnp.concatenate([p[i] for p in parts], axis=0) for i in range(5))
    new_Z = jnp.stack([za, zb], axis=-1).reshape(D, N)
    return new_Z, probs, eta_new, CDK_new


# speedup vs baseline: 1.0648x; 1.0648x over previous
import functools

import jax
import jax.numpy as jnp
import numpy as np
from jax import lax
from jax.experimental import pallas as pl
from jax.experimental.pallas import tpu as pltpu
from jax.experimental.pallas import tpu_sc as plsc

ZERO = 1e-06
ETA_VAR = 1.0
SGLD_A, SGLD_B, SGLD_C = 0.01, 1.0, 0.55

_K2_HI, _K2_LO = 64467757, 2916123636


def _u32(v):
    return jnp.uint32(np.uint32(v))


def _threefry_bits(cnt):
    ks0 = _K2_HI
    ks1 = _K2_LO
    ks2 = ks0 ^ ks1 ^ 0x1BD11BDA
    x0 = jnp.full_like(cnt, _u32(ks0))
    x1 = cnt + _u32(ks1)
    rots = ((13, 15, 26, 6), (17, 29, 16, 24))
    keysched = ((ks1, ks2, 1), (ks2, ks0, 2), (ks0, ks1, 3),
                (ks1, ks2, 4), (ks2, ks0, 5))
    for i, (ka, kb, c) in enumerate(keysched):
        for r in rots[i % 2]:
            x0 = x0 + x1
            x1 = (x1 << _u32(r)) | (x1 >> _u32(32 - r))
            x1 = x1 ^ x0
        x0 = x0 + _u32(ka)
        x1 = x1 + _u32((kb + c) & 0xFFFFFFFF)
    return x0 ^ x1


def _sc_gather(phi, words2d):
    R, CH = words2d.shape
    Dm = phi.shape[1]
    B = R * CH
    info = plsc.get_sparse_core_info()
    NW = info.num_cores * info.num_subcores
    n_ch = R // NW
    mesh = plsc.VectorSubcoreMesh(core_axis_name="c", subcore_axis_name="s")

    @functools.partial(
        pl.kernel,
        mesh=mesh,
        compiler_params=pltpu.CompilerParams(use_tc_tiling_on_sc=False),
        out_type=jax.ShapeDtypeStruct((B, Dm), jnp.float32),
        scratch_types=[
            pltpu.VMEM((n_ch, CH), jnp.int32),
            pltpu.VMEM((CH, Dm), jnp.float32),
            pltpu.VMEM((CH, Dm), jnp.float32),
            pltpu.SemaphoreType.DMA,
            pltpu.SemaphoreType.DMA,
        ],
    )
    def k(idx_hbm, table_hbm, out_hbm, idx_v, rows_a, rows_b, sem_a, sem_b):
        wid = lax.axis_index("s") * info.num_cores + lax.axis_index("c")
        pltpu.sync_copy(idx_hbm.at[pl.ds(wid * n_ch, n_ch)], idx_v)
        obase = wid * n_ch * CH

        def gather(j, buf, sem):
            return pltpu.make_async_copy(table_hbm.at[idx_v.at[j]], buf, sem)

        gather(0, rows_a, sem_a).start()

        def body(t, _):
            j0 = 2 * t
            j1 = j0 + 1
            out2 = out_hbm
            gather(j1, rows_b, sem_b).start()
            gather(j0, rows_a, sem_a).wait()
            pltpu.sync_copy(rows_a, out2.at[pl.ds(obase + j0 * CH, CH)])

            @pl.when(j0 + 2 < n_ch)
            def _():
                gather(j0 + 2, rows_a, sem_a).start()

            gather(j1, rows_b, sem_b).wait()
            pltpu.sync_copy(rows_b, out2.at[pl.ds(obase + j1 * CH, CH)])
            return 0

        lax.fori_loop(0, n_ch // 2, body, 0)
        if n_ch % 2:
            j = n_ch - 1
            gather(j, rows_a, sem_a).wait()
            pltpu.sync_copy(rows_a, out_hbm.at[pl.ds(obase + j * CH, CH)])

    return k(words2d, phi)


def _tc_main(eta, alpha2d, CDK, xi, phi_w2, doc0=0):
    D, K = eta.shape
    NH = 100
    N = 2 * NH
    BD = 16
    eps = SGLD_A * (SGLD_B + 0.0) ** (-SGLD_C)
    eps_half = np.float32(eps / 2.0)

    def body(eta_ref, alpha_ref, cdk_ref, xi_ref, phiw_ref,
             z_ref, probs_ref, etan_ref, cdkn_ref):
        i = pl.program_id(0)
        eta_b = eta_ref[...]
        m = jnp.max(eta_b, axis=-1, keepdims=True)
        ex = jnp.exp(eta_b - m)
        sm = ex / jnp.sum(ex, axis=-1, keepdims=True)
        prior = (alpha_ref[...] - eta_b) / np.float32(ETA_VAR)
        grad = cdk_ref[...] - np.float32(N) * sm
        eta_new = eta_b + eps_half * (prior + grad) + xi_ref[...]
        etan_ref[...] = eta_new

        eta2 = jnp.concatenate([eta_new, eta_new], axis=-1)
        pw = phiw_ref[...].reshape(BD, NH, 128)
        logits = eta2[:, None, :] + pw

        p = jnp.maximum(jnp.exp(jnp.clip(logits, -700.0, 700.0)),
                        np.float32(ZERO))
        sa = jnp.sum(p[:, :, :K], axis=-1, keepdims=True)
        sb = jnp.sum(p[:, :, K:], axis=-1, keepdims=True)
        denom = jnp.concatenate(
            [jnp.broadcast_to(sa, (BD, NH, K)),
             jnp.broadcast_to(sb, (BD, NH, K))], axis=-1)
        pr = p / denom
        probs_ref[:, :NH, :] = pr[:, :, :K]
        probs_ref[:, NH:, :] = pr[:, :, K:]

        jd = lax.broadcasted_iota(jnp.int32, (BD, NH, 128), 0)
        jn = lax.broadcasted_iota(jnp.int32, (BD, NH, 128), 1)
        jl = lax.broadcasted_iota(jnp.int32, (BD, NH, 128), 2)
        cnt = ((doc0 + i * BD + jd) * (N * K) + jn * K
               + (jl >> 6) * (NH * K) + (jl & 63)).astype(jnp.uint32)
        bits = _threefry_bits(cnt)
        fb = lax.bitcast_convert_type(
            (bits >> _u32(9)) | _u32(0x3F800000), jnp.float32
        ) - np.float32(1.0)
        u = jnp.maximum(
            np.float32(1e-9),
            fb * (np.float32(1.0) - np.float32(1e-9)) + np.float32(1e-9),
        )
        gm = logits - jnp.log(-jnp.log(u))
        ma = jnp.max(gm[:, :, :K], axis=-1, keepdims=True)
        mb = jnp.max(gm[:, :, K:], axis=-1, keepdims=True)
        mc = jnp.concatenate(
            [jnp.broadcast_to(ma, (BD, NH, K)),
             jnp.broadcast_to(mb, (BD, NH, K))], axis=-1)
        kio = jnp.bitwise_and(jl, 63)
        cand = jnp.where(gm == mc, kio, K)
        za = jnp.min(cand[:, :, :K], axis=-1)
        zb = jnp.min(cand[:, :, K:], axis=-1)
        z_ref[:, :NH] = za
        z_ref[:, NH:] = zb
        kio64 = lax.broadcasted_iota(jnp.int32, (BD, NH, K), 2)
        oh = ((za[:, :, None] == kio64).astype(jnp.float32)
              + (zb[:, :, None] == kio64).astype(jnp.float32))
        cdkn_ref[...] = jnp.sum(oh, axis=1)

    grid = D // BD
    return pl.pallas_call(
        body,
        grid=(grid,),
        in_specs=[
            pl.BlockSpec((BD, K), lambda i: (i, 0)),
            pl.BlockSpec((1, K), lambda i: (0, 0)),
            pl.BlockSpec((BD, K), lambda i: (i, 0)),
            pl.BlockSpec((BD, 1), lambda i: (i, 0)),
            pl.BlockSpec((BD * NH, 128), lambda i: (i, 0)),
        ],
        out_specs=[
            pl.BlockSpec((BD, N), lambda i: (i, 0)),
            pl.BlockSpec((BD, N, K), lambda i: (i, 0, 0)),
            pl.BlockSpec((BD, K), lambda i: (i, 0)),
            pl.BlockSpec((BD, K), lambda i: (i, 0)),
        ],
        out_shape=[
            jax.ShapeDtypeStruct((D, N), jnp.int32),
            jax.ShapeDtypeStruct((D, N, K), jnp.float32),
            jax.ShapeDtypeStruct((D, K), jnp.float32),
            jax.ShapeDtypeStruct((D, K), jnp.float32),
        ],
    )(eta, alpha2d, CDK, xi, phi_w2)


def kernel(eta, alpha, CDK, phi, words):
    D, K = eta.shape
    N = words.shape[1]
    eps = SGLD_A * (SGLD_B + 0.0) ** (-SGLD_C)
    k1 = jax.random.split(jax.random.key(42))[0]
    xi = jax.random.normal(k1, (D, 1), dtype=eta.dtype) * eps

    NH = N // 2
    words_pairs = jnp.transpose(words.reshape(D, 2, NH), (0, 2, 1))
    words2d = words_pairs.reshape((D * N) // 128, 128)
    phi_w2 = _sc_gather(phi, words2d).reshape((D * N * K) // 128, 128)
    new_Z, probs, eta_new, CDK_new = _tc_main(
        eta, alpha.reshape(1, K), CDK, xi, phi_w2)
    return new_Z, probs, eta_new, CDK_new


# speedup vs baseline: 1.0827x; 1.0168x over previous
import functools

import jax
import jax.numpy as jnp
import numpy as np
from jax import lax
from jax.experimental import pallas as pl
from jax.experimental.pallas import tpu as pltpu
from jax.experimental.pallas import tpu_sc as plsc

ZERO = 1e-06
ETA_VAR = 1.0
SGLD_A, SGLD_B, SGLD_C = 0.01, 1.0, 0.55

_K2_HI, _K2_LO = 64467757, 2916123636


def _u32(v):
    return jnp.uint32(np.uint32(v))


def _threefry_bits(cnt):
    ks0 = _K2_HI
    ks1 = _K2_LO
    ks2 = ks0 ^ ks1 ^ 0x1BD11BDA
    x0 = jnp.full_like(cnt, _u32(ks0))
    x1 = cnt + _u32(ks1)
    rots = ((13, 15, 26, 6), (17, 29, 16, 24))
    keysched = ((ks1, ks2, 1), (ks2, ks0, 2), (ks0, ks1, 3),
                (ks1, ks2, 4), (ks2, ks0, 5))
    for i, (ka, kb, c) in enumerate(keysched):
        for r in rots[i % 2]:
            x0 = x0 + x1
            x1 = (x1 << _u32(r)) | (x1 >> _u32(32 - r))
            x1 = x1 ^ x0
        x0 = x0 + _u32(ka)
        x1 = x1 + _u32((kb + c) & 0xFFFFFFFF)
    return x0 ^ x1


def _sc_gather(phi, words2d):
    R, CH = words2d.shape
    Dm = phi.shape[1]
    B = R * CH
    info = plsc.get_sparse_core_info()
    NW = info.num_cores * info.num_subcores
    n_ch = R // NW
    mesh = plsc.VectorSubcoreMesh(core_axis_name="c", subcore_axis_name="s")

    @functools.partial(
        pl.kernel,
        mesh=mesh,
        compiler_params=pltpu.CompilerParams(use_tc_tiling_on_sc=False),
        out_type=jax.ShapeDtypeStruct((B, Dm), jnp.float32),
        scratch_types=[
            pltpu.VMEM((n_ch, CH), jnp.int32),
            pltpu.VMEM((CH, Dm), jnp.float32),
            pltpu.VMEM((CH, Dm), jnp.float32),
            pltpu.SemaphoreType.DMA,
            pltpu.SemaphoreType.DMA,
        ],
    )
    def k(idx_hbm, table_hbm, out_hbm, idx_v, rows_a, rows_b, sem_a, sem_b):
        wid = lax.axis_index("s") * info.num_cores + lax.axis_index("c")
        pltpu.sync_copy(idx_hbm.at[pl.ds(wid * n_ch, n_ch)], idx_v)
        obase = wid * n_ch * CH

        def gather(j, buf, sem):
            return pltpu.make_async_copy(table_hbm.at[idx_v.at[j]], buf, sem)

        gather(0, rows_a, sem_a).start()

        def body(t, _):
            j0 = 2 * t
            j1 = j0 + 1
            gather(j1, rows_b, sem_b).start()
            gather(j0, rows_a, sem_a).wait()
            pltpu.sync_copy(rows_a, out_hbm.at[pl.ds(obase + j0 * CH, CH)])

            @pl.when(j0 + 2 < n_ch)
            def _():
                gather(j0 + 2, rows_a, sem_a).start()

            gather(j1, rows_b, sem_b).wait()
            pltpu.sync_copy(rows_b, out_hbm.at[pl.ds(obase + j1 * CH, CH)])
            return 0

        lax.fori_loop(0, n_ch // 2, body, 0)
        if n_ch % 2:
            j = n_ch - 1
            gather(j, rows_a, sem_a).wait()
            pltpu.sync_copy(rows_a, out_hbm.at[pl.ds(obase + j * CH, CH)])

    return k(words2d, phi)


def _tc_main(eta, alpha2d, CDK, xi, phi_w2):
    D, K = eta.shape
    NH = 100
    N = 2 * NH
    BD = 16
    eps = SGLD_A * (SGLD_B + 0.0) ** (-SGLD_C)
    eps_half = np.float32(eps / 2.0)

    def body(eta_ref, alpha_ref, cdk_ref, xi_ref, phiw_ref,
             za_ref, zb_ref, probs_ref, etan_ref, cdkn_ref):
        i = pl.program_id(0)
        eta_b = eta_ref[...]
        m = jnp.max(eta_b, axis=-1, keepdims=True)
        ex = jnp.exp(eta_b - m)
        sm = ex / jnp.sum(ex, axis=-1, keepdims=True)
        prior = (alpha_ref[...] - eta_b) / np.float32(ETA_VAR)
        grad = cdk_ref[...] - np.float32(N) * sm
        eta_new = eta_b + eps_half * (prior + grad) + xi_ref[...]
        etan_ref[...] = eta_new

        eta2 = jnp.concatenate([eta_new, eta_new], axis=-1)
        pw = phiw_ref[...].reshape(BD, NH, 128)
        logits = eta2[:, None, :] + pw

        p = jnp.maximum(jnp.exp(jnp.clip(logits, -700.0, 700.0)),
                        np.float32(ZERO))
        sa = jnp.sum(p[:, :, :K], axis=-1, keepdims=True)
        sb = jnp.sum(p[:, :, K:], axis=-1, keepdims=True)
        denom = jnp.concatenate(
            [jnp.broadcast_to(sa, (BD, NH, K)),
             jnp.broadcast_to(sb, (BD, NH, K))], axis=-1)
        probs_ref[...] = (p / denom).reshape(BD * NH, 128)

        jrow = (lax.broadcasted_iota(jnp.int32, (BD, NH, 128), 0) * NH
                + lax.broadcasted_iota(jnp.int32, (BD, NH, 128), 1))
        jl = lax.broadcasted_iota(jnp.int32, (BD, NH, 128), 2)
        cnt = ((i * (BD * NH * 128)) + jrow * 128 + jl).astype(jnp.uint32)
        bits = _threefry_bits(cnt)
        fb = lax.bitcast_convert_type(
            (bits >> _u32(9)) | _u32(0x3F800000), jnp.float32
        ) - np.float32(1.0)
        u = jnp.maximum(
            np.float32(1e-9),
            fb * (np.float32(1.0) - np.float32(1e-9)) + np.float32(1e-9),
        )
        gm = logits - jnp.log(-jnp.log(u))
        ma = jnp.max(gm[:, :, :K], axis=-1, keepdims=True)
        mb = jnp.max(gm[:, :, K:], axis=-1, keepdims=True)
        mc = jnp.concatenate(
            [jnp.broadcast_to(ma, (BD, NH, K)),
             jnp.broadcast_to(mb, (BD, NH, K))], axis=-1)
        kio = jnp.bitwise_and(jl, 63)
        cand = jnp.where(gm == mc, kio, K)
        za = jnp.min(cand[:, :, :K], axis=-1)
        zb = jnp.min(cand[:, :, K:], axis=-1)
        za_ref[...] = za
        zb_ref[...] = zb
        kio64 = lax.broadcasted_iota(jnp.int32, (BD, NH, K), 2)
        oh = ((za[:, :, None] == kio64).astype(jnp.float32)
              + (zb[:, :, None] == kio64).astype(jnp.float32))
        cdkn_ref[...] = jnp.sum(oh, axis=1)

    grid = D // BD
    return pl.pallas_call(
        body,
        grid=(grid,),
        in_specs=[
            pl.BlockSpec((BD, K), lambda i: (i, 0)),
            pl.BlockSpec((1, K), lambda i: (0, 0)),
            pl.BlockSpec((BD, K), lambda i: (i, 0)),
            pl.BlockSpec((BD, 1), lambda i: (i, 0)),
            pl.BlockSpec((BD * NH, 128), lambda i: (i, 0)),
        ],
        out_specs=[
            pl.BlockSpec((BD, NH), lambda i: (i, 0)),
            pl.BlockSpec((BD, NH), lambda i: (i, 0)),
            pl.BlockSpec((BD * NH, 128), lambda i: (i, 0)),
            pl.BlockSpec((BD, K), lambda i: (i, 0)),
            pl.BlockSpec((BD, K), lambda i: (i, 0)),
        ],
        out_shape=[
            jax.ShapeDtypeStruct((D, NH), jnp.int32),
            jax.ShapeDtypeStruct((D, NH), jnp.int32),
            jax.ShapeDtypeStruct((D * NH, 128), jnp.float32),
            jax.ShapeDtypeStruct((D, K), jnp.float32),
            jax.ShapeDtypeStruct((D, K), jnp.float32),
        ],
    )(eta, alpha2d, CDK, xi, phi_w2)


def kernel(eta, alpha, CDK, phi, words):
    D, K = eta.shape
    N = words.shape[1]
    eps = SGLD_A * (SGLD_B + 0.0) ** (-SGLD_C)
    k1 = jax.random.split(jax.random.key(42))[0]
    xi = jax.random.normal(k1, (D, 1), dtype=eta.dtype) * eps

    words2d = words.reshape((D * N) // 128, 128)
    phi_w2 = _sc_gather(phi, words2d).reshape((D * N * K) // 128, 128)

    za, zb, probs2, eta_new, CDK_new = _tc_main(
        eta, alpha.reshape(1, K), CDK, xi, phi_w2
    )
    new_Z = jnp.stack([za, zb], axis=-1).reshape(D, N)
    probs = probs2.reshape(D, N, K)
    return new_Z, probs, eta_new, CDK_new


# speedup vs baseline: 1.0977x; 1.0139x over previous
import functools

import jax
import jax.numpy as jnp
import numpy as np
from jax import lax
from jax.experimental import pallas as pl
from jax.experimental.pallas import tpu as pltpu
from jax.experimental.pallas import tpu_sc as plsc

ZERO = 1e-06
ETA_VAR = 1.0
SGLD_A, SGLD_B, SGLD_C = 0.01, 1.0, 0.55

_K2_HI, _K2_LO = 64467757, 2916123636


def _u32(v):
    return jnp.uint32(np.uint32(v))


def _threefry_bits(cnt):
    ks0 = _K2_HI
    ks1 = _K2_LO
    ks2 = ks0 ^ ks1 ^ 0x1BD11BDA
    x0 = jnp.full_like(cnt, _u32(ks0))
    x1 = cnt + _u32(ks1)
    rots = ((13, 15, 26, 6), (17, 29, 16, 24))
    keysched = ((ks1, ks2, 1), (ks2, ks0, 2), (ks0, ks1, 3),
                (ks1, ks2, 4), (ks2, ks0, 5))
    for i, (ka, kb, c) in enumerate(keysched):
        for r in rots[i % 2]:
            x0 = x0 + x1
            x1 = (x1 << _u32(r)) | (x1 >> _u32(32 - r))
            x1 = x1 ^ x0
        x0 = x0 + _u32(ka)
        x1 = x1 + _u32((kb + c) & 0xFFFFFFFF)
    return x0 ^ x1


def _sc_gather(phi, words2d):
    R, CH = words2d.shape
    Dm = phi.shape[1]
    B = R * CH
    info = plsc.get_sparse_core_info()
    NW = info.num_cores * info.num_subcores
    n_ch = R // NW
    mesh = plsc.VectorSubcoreMesh(core_axis_name="c", subcore_axis_name="s")

    @functools.partial(
        pl.kernel,
        mesh=mesh,
        compiler_params=pltpu.CompilerParams(use_tc_tiling_on_sc=False),
        out_type=jax.ShapeDtypeStruct((B, Dm), jnp.float32),
        scratch_types=[
            pltpu.VMEM((n_ch, CH), jnp.int32),
            pltpu.VMEM((CH, Dm), jnp.float32),
            pltpu.VMEM((CH, Dm), jnp.float32),
            pltpu.SemaphoreType.DMA,
            pltpu.SemaphoreType.DMA,
        ],
    )
    def k(idx_hbm, table_hbm, out_hbm, idx_v, rows_a, rows_b, sem_a, sem_b):
        wid = lax.axis_index("s") * info.num_cores + lax.axis_index("c")
        pltpu.sync_copy(idx_hbm.at[pl.ds(wid * n_ch, n_ch)], idx_v)
        obase = wid * n_ch * CH

        def gather(j, buf, sem):
            return pltpu.make_async_copy(table_hbm.at[idx_v.at[j]], buf, sem)

        gather(0, rows_a, sem_a).start()

        def body(t, _):
            j0 = 2 * t
            j1 = j0 + 1
            gather(j1, rows_b, sem_b).start()
            gather(j0, rows_a, sem_a).wait()
            pltpu.sync_copy(rows_a, out_hbm.at[pl.ds(obase + j0 * CH, CH)])

            @pl.when(j0 + 2 < n_ch)
            def _():
                gather(j0 + 2, rows_a, sem_a).start()

            gather(j1, rows_b, sem_b).wait()
            pltpu.sync_copy(rows_b, out_hbm.at[pl.ds(obase + j1 * CH, CH)])
            return 0

        lax.fori_loop(0, n_ch // 2, body, 0)
        if n_ch % 2:
            j = n_ch - 1
            gather(j, rows_a, sem_a).wait()
            pltpu.sync_copy(rows_a, out_hbm.at[pl.ds(obase + j * CH, CH)])

    return k(words2d, phi)


def _tc_main(eta, alpha2d, CDK, xi, phi_w2):
    D, K = eta.shape
    NH = 100
    N = 2 * NH
    BD = 32
    eps = SGLD_A * (SGLD_B + 0.0) ** (-SGLD_C)
    eps_half = np.float32(eps / 2.0)

    def body(eta_ref, alpha_ref, cdk_ref, xi_ref, phiw_ref,
             za_ref, zb_ref, probs_ref, etan_ref, cdkn_ref):
        i = pl.program_id(0)
        eta_b = eta_ref[...]
        m = jnp.max(eta_b, axis=-1, keepdims=True)
        ex = jnp.exp(eta_b - m)
        sm = ex / jnp.sum(ex, axis=-1, keepdims=True)
        prior = (alpha_ref[...] - eta_b) / np.float32(ETA_VAR)
        grad = cdk_ref[...] - np.float32(N) * sm
        eta_new = eta_b + eps_half * (prior + grad) + xi_ref[...]
        etan_ref[...] = eta_new

        eta2 = jnp.concatenate([eta_new, eta_new], axis=-1)
        pw = phiw_ref[...].reshape(BD, NH, 128)
        logits = eta2[:, None, :] + pw

        p = jnp.maximum(jnp.exp(jnp.clip(logits, -700.0, 700.0)),
                        np.float32(ZERO))
        sa = jnp.sum(p[:, :, :K], axis=-1, keepdims=True)
        sb = jnp.sum(p[:, :, K:], axis=-1, keepdims=True)
        denom = jnp.concatenate(
            [jnp.broadcast_to(sa, (BD, NH, K)),
             jnp.broadcast_to(sb, (BD, NH, K))], axis=-1)
        probs_ref[...] = (p / denom).reshape(BD * NH, 128)

        jrow = (lax.broadcasted_iota(jnp.int32, (BD, NH, 128), 0) * NH
                + lax.broadcasted_iota(jnp.int32, (BD, NH, 128), 1))
        jl = lax.broadcasted_iota(jnp.int32, (BD, NH, 128), 2)
        cnt = ((i * (BD * NH * 128)) + jrow * 128 + jl).astype(jnp.uint32)
        bits = _threefry_bits(cnt)
        fb = lax.bitcast_convert_type(
            (bits >> _u32(9)) | _u32(0x3F800000), jnp.float32
        ) - np.float32(1.0)
        u = jnp.maximum(
            np.float32(1e-9),
            fb * (np.float32(1.0) - np.float32(1e-9)) + np.float32(1e-9),
        )
        gm = logits - jnp.log(-jnp.log(u))
        ma = jnp.max(gm[:, :, :K], axis=-1, keepdims=True)
        mb = jnp.max(gm[:, :, K:], axis=-1, keepdims=True)
        mc = jnp.concatenate(
            [jnp.broadcast_to(ma, (BD, NH, K)),
             jnp.broadcast_to(mb, (BD, NH, K))], axis=-1)
        kio = jnp.bitwise_and(jl, 63)
        cand = jnp.where(gm == mc, kio, K)
        za = jnp.min(cand[:, :, :K], axis=-1)
        zb = jnp.min(cand[:, :, K:], axis=-1)
        za_ref[...] = za
        zb_ref[...] = zb
        kio64 = lax.broadcasted_iota(jnp.int32, (BD, NH, K), 2)
        oh = ((za[:, :, None] == kio64).astype(jnp.float32)
              + (zb[:, :, None] == kio64).astype(jnp.float32))
        cdkn_ref[...] = jnp.sum(oh, axis=1)

    grid = D // BD
    return pl.pallas_call(
        body,
        grid=(grid,),
        in_specs=[
            pl.BlockSpec((BD, K), lambda i: (i, 0)),
            pl.BlockSpec((1, K), lambda i: (0, 0)),
            pl.BlockSpec((BD, K), lambda i: (i, 0)),
            pl.BlockSpec((BD, 1), lambda i: (i, 0)),
            pl.BlockSpec((BD * NH, 128), lambda i: (i, 0)),
        ],
        out_specs=[
            pl.BlockSpec((BD, NH), lambda i: (i, 0)),
            pl.BlockSpec((BD, NH), lambda i: (i, 0)),
            pl.BlockSpec((BD * NH, 128), lambda i: (i, 0)),
            pl.BlockSpec((BD, K), lambda i: (i, 0)),
            pl.BlockSpec((BD, K), lambda i: (i, 0)),
        ],
        out_shape=[
            jax.ShapeDtypeStruct((D, NH), jnp.int32),
            jax.ShapeDtypeStruct((D, NH), jnp.int32),
            jax.ShapeDtypeStruct((D * NH, 128), jnp.float32),
            jax.ShapeDtypeStruct((D, K), jnp.float32),
            jax.ShapeDtypeStruct((D, K), jnp.float32),
        ],
    )(eta, alpha2d, CDK, xi, phi_w2)


def kernel(eta, alpha, CDK, phi, words):
    D, K = eta.shape
    N = words.shape[1]
    eps = SGLD_A * (SGLD_B + 0.0) ** (-SGLD_C)
    k1 = jax.random.split(jax.random.key(42))[0]
    xi = jax.random.normal(k1, (D, 1), dtype=eta.dtype) * eps

    words2d = words.reshape((D * N) // 128, 128)
    phi_w2 = _sc_gather(phi, words2d).reshape((D * N * K) // 128, 128)

    za, zb, probs2, eta_new, CDK_new = _tc_main(
        eta, alpha.reshape(1, K), CDK, xi, phi_w2
    )
    new_Z = jnp.stack([za, zb], axis=-1).reshape(D, N)
    probs = probs2.reshape(D, N, K)
    return new_Z, probs, eta_new, CDK_new


# speedup vs baseline: 1.1075x; 1.0089x over previous
import functools

import jax
import jax.numpy as jnp
import numpy as np
from jax import lax
from jax.experimental import pallas as pl
from jax.experimental.pallas import tpu as pltpu
from jax.experimental.pallas import tpu_sc as plsc

ZERO = 1e-06
ETA_VAR = 1.0
SGLD_A, SGLD_B, SGLD_C = 0.01, 1.0, 0.55

_K2_HI, _K2_LO = 64467757, 2916123636


def _u32(v):
    return jnp.uint32(np.uint32(v))


def _threefry_bits(cnt):
    ks0 = _K2_HI
    ks1 = _K2_LO
    ks2 = ks0 ^ ks1 ^ 0x1BD11BDA
    x0 = jnp.full_like(cnt, _u32(ks0))
    x1 = cnt + _u32(ks1)
    rots = ((13, 15, 26, 6), (17, 29, 16, 24))
    keysched = ((ks1, ks2, 1), (ks2, ks0, 2), (ks0, ks1, 3),
                (ks1, ks2, 4), (ks2, ks0, 5))
    for i, (ka, kb, c) in enumerate(keysched):
        for r in rots[i % 2]:
            x0 = x0 + x1
            x1 = (x1 << _u32(r)) | (x1 >> _u32(32 - r))
            x1 = x1 ^ x0
        x0 = x0 + _u32(ka)
        x1 = x1 + _u32((kb + c) & 0xFFFFFFFF)
    return x0 ^ x1


def _sc_gather(phi, words2d):
    R, CH = words2d.shape
    Dm = phi.shape[1]
    B = R * CH
    info = plsc.get_sparse_core_info()
    NW = info.num_cores * info.num_subcores
    n_ch = R // NW
    mesh = plsc.VectorSubcoreMesh(core_axis_name="c", subcore_axis_name="s")

    @functools.partial(
        pl.kernel,
        mesh=mesh,
        compiler_params=pltpu.CompilerParams(use_tc_tiling_on_sc=False),
        out_type=jax.ShapeDtypeStruct((B, Dm), jnp.float32),
        scratch_types=[
            pltpu.VMEM((n_ch, CH), jnp.int32),
            pltpu.VMEM((CH, Dm), jnp.float32),
            pltpu.VMEM((CH, Dm), jnp.float32),
            pltpu.SemaphoreType.DMA,
            pltpu.SemaphoreType.DMA,
        ],
    )
    def k(idx_hbm, table_hbm, out_hbm, idx_v, rows_a, rows_b, sem_a, sem_b):
        wid = lax.axis_index("s") * info.num_cores + lax.axis_index("c")
        pltpu.sync_copy(idx_hbm.at[pl.ds(wid * n_ch, n_ch)], idx_v)
        obase = wid * n_ch * CH

        def gather(j, buf, sem):
            return pltpu.make_async_copy(table_hbm.at[idx_v.at[j]], buf, sem)

        gather(0, rows_a, sem_a).start()

        def body(t, _):
            j0 = 2 * t
            j1 = j0 + 1
            gather(j1, rows_b, sem_b).start()
            gather(j0, rows_a, sem_a).wait()
            pltpu.sync_copy(rows_a, out_hbm.at[pl.ds(obase + j0 * CH, CH)])

            @pl.when(j0 + 2 < n_ch)
            def _():
                gather(j0 + 2, rows_a, sem_a).start()

            gather(j1, rows_b, sem_b).wait()
            pltpu.sync_copy(rows_b, out_hbm.at[pl.ds(obase + j1 * CH, CH)])
            return 0

        lax.fori_loop(0, n_ch // 2, body, 0)
        if n_ch % 2:
            j = n_ch - 1
            gather(j, rows_a, sem_a).wait()
            pltpu.sync_copy(rows_a, out_hbm.at[pl.ds(obase + j * CH, CH)])

    return k(words2d, phi)


def _tc_main(eta, alpha2d, CDK, xi, phi_w2):
    D, K = eta.shape
    NH = 100
    N = 2 * NH
    BD = 64
    eps = SGLD_A * (SGLD_B + 0.0) ** (-SGLD_C)
    eps_half = np.float32(eps / 2.0)

    def body(eta_ref, alpha_ref, cdk_ref, xi_ref, phiw_ref,
             za_ref, zb_ref, probs_ref, etan_ref, cdkn_ref):
        i = pl.program_id(0)
        eta_b = eta_ref[...]
        m = jnp.max(eta_b, axis=-1, keepdims=True)
        ex = jnp.exp(eta_b - m)
        sm = ex / jnp.sum(ex, axis=-1, keepdims=True)
        prior = (alpha_ref[...] - eta_b) / np.float32(ETA_VAR)
        grad = cdk_ref[...] - np.float32(N) * sm
        eta_new = eta_b + eps_half * (prior + grad) + xi_ref[...]
        etan_ref[...] = eta_new

        eta2 = jnp.concatenate([eta_new, eta_new], axis=-1)
        pw = phiw_ref[...].reshape(BD, NH, 128)
        logits = eta2[:, None, :] + pw

        p = jnp.maximum(jnp.exp(jnp.clip(logits, -700.0, 700.0)),
                        np.float32(ZERO))
        sa = jnp.sum(p[:, :, :K], axis=-1, keepdims=True)
        sb = jnp.sum(p[:, :, K:], axis=-1, keepdims=True)
        denom = jnp.concatenate(
            [jnp.broadcast_to(sa, (BD, NH, K)),
             jnp.broadcast_to(sb, (BD, NH, K))], axis=-1)
        probs_ref[...] = (p / denom).reshape(BD * NH, 128)

        jrow = (lax.broadcasted_iota(jnp.int32, (BD, NH, 128), 0) * NH
                + lax.broadcasted_iota(jnp.int32, (BD, NH, 128), 1))
        jl = lax.broadcasted_iota(jnp.int32, (BD, NH, 128), 2)
        cnt = ((i * (BD * NH * 128)) + jrow * 128 + jl).astype(jnp.uint32)
        bits = _threefry_bits(cnt)
        fb = lax.bitcast_convert_type(
            (bits >> _u32(9)) | _u32(0x3F800000), jnp.float32
        ) - np.float32(1.0)
        u = jnp.maximum(
            np.float32(1e-9),
            fb * (np.float32(1.0) - np.float32(1e-9)) + np.float32(1e-9),
        )
        gm = logits - jnp.log(-jnp.log(u))
        ma = jnp.max(gm[:, :, :K], axis=-1, keepdims=True)
        mb = jnp.max(gm[:, :, K:], axis=-1, keepdims=True)
        mc = jnp.concatenate(
            [jnp.broadcast_to(ma, (BD, NH, K)),
             jnp.broadcast_to(mb, (BD, NH, K))], axis=-1)
        kio = jnp.bitwise_and(jl, 63)
        cand = jnp.where(gm == mc, kio, K)
        za = jnp.min(cand[:, :, :K], axis=-1)
        zb = jnp.min(cand[:, :, K:], axis=-1)
        za_ref[...] = za
        zb_ref[...] = zb
        kio64 = lax.broadcasted_iota(jnp.int32, (BD, NH, K), 2)
        oh = ((za[:, :, None] == kio64).astype(jnp.float32)
              + (zb[:, :, None] == kio64).astype(jnp.float32))
        cdkn_ref[...] = jnp.sum(oh, axis=1)

    grid = D // BD
    return pl.pallas_call(
        body,
        grid=(grid,),
        in_specs=[
            pl.BlockSpec((BD, K), lambda i: (i, 0)),
            pl.BlockSpec((1, K), lambda i: (0, 0)),
            pl.BlockSpec((BD, K), lambda i: (i, 0)),
            pl.BlockSpec((BD, 1), lambda i: (i, 0)),
            pl.BlockSpec((BD * NH, 128), lambda i: (i, 0)),
        ],
        out_specs=[
            pl.BlockSpec((BD, NH), lambda i: (i, 0)),
            pl.BlockSpec((BD, NH), lambda i: (i, 0)),
            pl.BlockSpec((BD * NH, 128), lambda i: (i, 0)),
            pl.BlockSpec((BD, K), lambda i: (i, 0)),
            pl.BlockSpec((BD, K), lambda i: (i, 0)),
        ],
        out_shape=[
            jax.ShapeDtypeStruct((D, NH), jnp.int32),
            jax.ShapeDtypeStruct((D, NH), jnp.int32),
            jax.ShapeDtypeStruct((D * NH, 128), jnp.float32),
            jax.ShapeDtypeStruct((D, K), jnp.float32),
            jax.ShapeDtypeStruct((D, K), jnp.float32),
        ],
    )(eta, alpha2d, CDK, xi, phi_w2)


def kernel(eta, alpha, CDK, phi, words):
    D, K = eta.shape
    N = words.shape[1]
    eps = SGLD_A * (SGLD_B + 0.0) ** (-SGLD_C)
    k1 = jax.random.split(jax.random.key(42))[0]
    xi = jax.random.normal(k1, (D, 1), dtype=eta.dtype) * eps

    words2d = words.reshape((D * N) // 128, 128)
    phi_w2 = _sc_gather(phi, words2d).reshape((D * N * K) // 128, 128)

    za, zb, probs2, eta_new, CDK_new = _tc_main(
        eta, alpha.reshape(1, K), CDK, xi, phi_w2
    )
    new_Z = jnp.stack([za, zb], axis=-1).reshape(D, N)
    probs = probs2.reshape(D, N, K)
    return new_Z, probs, eta_new, CDK_new
